# baseline (speedup 1.0000x reference)
import functools

import jax
import jax.numpy as jnp
from jax import lax
from jax.experimental import pallas as pl
from jax.experimental.pallas import tpu as pltpu

B, S, D_IN = 16, 512, 32
D_MODEL, H, D_K = 1024, 16, 64
U_TOP = 31
N_TAB = 48

f32 = jnp.float32
bf16 = jnp.bfloat16


def _dot(a, b, dims):
    return lax.dot_general(a, b, (dims, ((), ())), preferred_element_type=f32)


def _body(values_ref, times_ref, pos_ref, wval_ref, tabs_ref, wc_ref,
          wq_ref, wk_ref, wv_ref, bval_ref, bconv_ref, bq_ref, bk_ref, bv_ref,
          out_ref, q_scr, k_scr, v_scr, vmean_scr, kc_scr):
    h = pl.program_id(1)

    @pl.when(h == 0)
    def _qkv():
        col = lax.broadcasted_iota(jnp.int32, (S, N_TAB), 1)
        t0 = times_ref[0, :, 0:1]
        t1 = times_ref[0, :, 1:2]
        t2 = times_ref[0, :, 2:3]
        oh = ((col == t0) | (col == t1 + 24) | (col == t2 + 31)).astype(bf16)
        time_emb = _dot(oh, tabs_ref[...], (((1,), (0,))))
        value_emb = _dot(values_ref[0], wval_ref[...], (((1,), (0,))))
        combined = value_emb + bval_ref[...] + pos_ref[...] + time_emb
        cb = combined.astype(bf16)
        zrow = jnp.zeros((1, D_MODEL), bf16)
        c_prev = jnp.concatenate([zrow, cb[:-1]], axis=0)
        c_next = jnp.concatenate([cb[1:], zrow], axis=0)
        x = (_dot(c_prev, wc_ref[0], (((1,), (0,))))
             + _dot(cb, wc_ref[1], (((1,), (0,))))
             + _dot(c_next, wc_ref[2], (((1,), (0,))))
             + bconv_ref[...])
        x = jnp.where(x > 0, x, jnp.exp(x) - 1.0)
        xb = x.astype(bf16)
        q = _dot(xb, wq_ref[...], (((1,), (0,)))) + bq_ref[...]
        k = _dot(xb, wk_ref[...], (((1,), (0,)))) + bk_ref[...]
        v = _dot(xb, wv_ref[...], (((1,), (0,)))) + bv_ref[...]
        for hh in range(H):
            sl = slice(hh * D_K, (hh + 1) * D_K)
            q_scr[hh] = q[:, sl].astype(bf16)
            k_scr[hh] = k[:, sl].astype(bf16)
            v_scr[hh] = v[:, sl].astype(bf16)
            vmean_scr[hh:hh + 1] = jnp.mean(v[:, sl], axis=0, keepdims=True)
            kc_scr[hh] = (k[:, sl]
                          - jnp.mean(k[:, sl], axis=0, keepdims=True)).astype(bf16)

    def _head(hh):
        qb = q_scr[hh]
        kb = k_scr[hh]
        vb = v_scr[hh]

        s_cen = _dot(qb, kc_scr[hh], (((1,), (1,))))
        m_col = jnp.max(s_cen, axis=1, keepdims=True)
        m_row = jnp.transpose(m_col, (1, 0))
        c_gt = (m_col > m_row).astype(bf16)
        ones_row = jnp.ones((8, S), bf16)
        rank_row = _dot(ones_row, c_gt, (((1,), (0,))))[0:1]
        p_iota = lax.broadcasted_iota(jnp.int32, (U_TOP + 1, S), 0).astype(f32)
        p_sel = ((rank_row == p_iota)
                 & (p_iota < float(U_TOP))).astype(bf16)

        top_q = _dot(p_sel, qb, (((1,), (0,))))
        s_sel = _dot(top_q.astype(bf16), kb, (((1,), (1,)))) * (1.0 / 8.0)
        s_sel = s_sel - jnp.max(s_sel, axis=1, keepdims=True)
        e_sel = jnp.exp(s_sel)
        attn = e_sel * (1.0 / jnp.sum(e_sel, axis=1, keepdims=True))
        out_sel = _dot(attn.astype(bf16), vb, (((1,), (0,))))

        v_mean = vmean_scr[pl.ds(hh, 1)]
        delta = (out_sel - v_mean).astype(bf16)
        return v_mean + _dot(p_sel, delta, (((0,), (0,))))

    out_ref[0, :, 0:D_K] = _head(4 * h)
    out_ref[0, :, D_K:2 * D_K] = _head(4 * h + 1)
    out_ref[0, :, 2 * D_K:3 * D_K] = _head(4 * h + 2)
    out_ref[0, :, 3 * D_K:4 * D_K] = _head(4 * h + 3)


@jax.jit
def kernel(values, times, W_val, b_val, hour_emb, week_emb, month_emb,
           pos_emb, W_conv, b_conv, Wq, bq, Wk, bk, Wv, bv):
    tabs = jnp.zeros((N_TAB, D_MODEL), f32)
    tabs = tabs.at[0:24].set(hour_emb).at[24:31].set(week_emb).at[31:43].set(month_emb)
    args = (
        values.astype(bf16),
        times.astype(jnp.int32),
        pos_emb[0, :S, :],
        W_val.astype(bf16),
        tabs.astype(bf16),
        W_conv.astype(bf16),
        Wq.astype(bf16),
        Wk.astype(bf16),
        Wv.astype(bf16),
        b_val.reshape(1, D_MODEL),
        b_conv.reshape(1, D_MODEL),
        bq.reshape(1, D_MODEL),
        bk.reshape(1, D_MODEL),
        bv.reshape(1, D_MODEL),
    )
    full = lambda *dims: pl.BlockSpec(dims, lambda b, h: (0,) * len(dims))
    grid_spec = pltpu.PrefetchScalarGridSpec(
        num_scalar_prefetch=0,
        grid=(B, H // 4),
        in_specs=[
            pl.BlockSpec((1, S, D_IN), lambda b, h: (b, 0, 0)),
            pl.BlockSpec((1, S, 3), lambda b, h: (b, 0, 0)),
            full(S, D_MODEL),
            full(D_IN, D_MODEL),
            full(N_TAB, D_MODEL),
            full(3, D_MODEL, D_MODEL),
            full(D_MODEL, D_MODEL),
            full(D_MODEL, D_MODEL),
            full(D_MODEL, D_MODEL),
            full(1, D_MODEL),
            full(1, D_MODEL),
            full(1, D_MODEL),
            full(1, D_MODEL),
            full(1, D_MODEL),
        ],
        out_specs=pl.BlockSpec((1, S, 4 * D_K), lambda b, h: (b, 0, h)),
        scratch_shapes=[
            pltpu.VMEM((H, S, D_K), bf16),
            pltpu.VMEM((H, S, D_K), bf16),
            pltpu.VMEM((H, S, D_K), bf16),
            pltpu.VMEM((H, D_K), f32),
            pltpu.VMEM((H, S, D_K), bf16),
        ],
    )
    return pl.pallas_call(
        _body,
        grid_spec=grid_spec,
        out_shape=jax.ShapeDtypeStruct((B, S, D_MODEL), f32),
        compiler_params=pltpu.CompilerParams(
            dimension_semantics=("parallel", "arbitrary"),
        ),
    )(*args)


# speedup vs baseline: 1.0573x; 1.0573x over previous
import functools

import jax
import jax.numpy as jnp
from jax import lax
from jax.experimental import pallas as pl
from jax.experimental.pallas import tpu as pltpu

B, S, D_IN = 16, 512, 32
D_MODEL, H, D_K = 1024, 16, 64
U_TOP = 31
N_TAB = 48

f32 = jnp.float32
bf16 = jnp.bfloat16


def _dot(a, b, dims):
    return lax.dot_general(a, b, (dims, ((), ())), preferred_element_type=f32)


def _body(values_ref, times_ref, pos_ref, wval_ref, tabs_ref, wc_ref,
          wq_ref, wk_ref, wv_ref, bval_ref, bconv_ref, bq_ref, bk_ref, bv_ref,
          out_ref, q_scr, k_scr, v_scr, vmean_scr, kc_scr):
    h = pl.program_id(1)

    @pl.when(h == 0)
    def _qkv():
        col = lax.broadcasted_iota(jnp.int32, (S, N_TAB), 1)
        t0 = times_ref[0, :, 0:1]
        t1 = times_ref[0, :, 1:2]
        t2 = times_ref[0, :, 2:3]
        oh = ((col == t0) | (col == t1 + 24) | (col == t2 + 31)).astype(bf16)
        time_emb = _dot(oh, tabs_ref[...], (((1,), (0,))))
        value_emb = _dot(values_ref[0], wval_ref[...], (((1,), (0,))))
        combined = value_emb + bval_ref[...] + pos_ref[...] + time_emb
        cb = combined.astype(bf16)
        zrow = jnp.zeros((1, D_MODEL), bf16)
        c_prev = jnp.concatenate([zrow, cb[:-1]], axis=0)
        c_next = jnp.concatenate([cb[1:], zrow], axis=0)
        x = (_dot(c_prev, wc_ref[0], (((1,), (0,))))
             + _dot(cb, wc_ref[1], (((1,), (0,))))
             + _dot(c_next, wc_ref[2], (((1,), (0,))))
             + bconv_ref[...])
        x = jnp.where(x > 0, x, jnp.exp(x) - 1.0)
        xb = x.astype(bf16)
        q = _dot(xb, wq_ref[...], (((1,), (0,)))) + bq_ref[...]
        k = _dot(xb, wk_ref[...], (((1,), (0,)))) + bk_ref[...]
        v = _dot(xb, wv_ref[...], (((1,), (0,)))) + bv_ref[...]
        for hh in range(H):
            sl = slice(hh * D_K, (hh + 1) * D_K)
            q_scr[hh] = q[:, sl].astype(bf16)
            k_scr[hh] = k[:, sl].astype(bf16)
            v_scr[hh] = v[:, sl].astype(bf16)
            vmean_scr[hh:hh + 1] = jnp.mean(v[:, sl], axis=0, keepdims=True)
            kc_scr[hh] = (k[:, sl]
                          - jnp.mean(k[:, sl], axis=0, keepdims=True)).astype(bf16)

    def _head(hh):
        qb = q_scr[hh]
        kb = k_scr[hh]
        vb = v_scr[hh]

        s_cen = _dot(qb, kc_scr[hh], (((1,), (1,))))
        m_col = jnp.max(s_cen, axis=1, keepdims=True)
        m_row = jnp.transpose(m_col, (1, 0))
        c_gt = (m_col > m_row).astype(f32)
        rank_row = jnp.sum(c_gt, axis=0, keepdims=True)
        p_iota = lax.broadcasted_iota(jnp.int32, (U_TOP + 1, S), 0).astype(f32)
        p_sel = ((rank_row == p_iota)
                 & (p_iota < float(U_TOP))).astype(bf16)

        top_q = _dot(p_sel, qb, (((1,), (0,))))
        s_sel = _dot(top_q.astype(bf16), kb, (((1,), (1,)))) * (1.0 / 8.0)
        s_sel = s_sel - jnp.max(s_sel, axis=1, keepdims=True)
        e_sel = jnp.exp(s_sel)
        attn = e_sel * (1.0 / jnp.sum(e_sel, axis=1, keepdims=True))
        out_sel = _dot(attn.astype(bf16), vb, (((1,), (0,))))

        v_mean = vmean_scr[pl.ds(hh, 1)]
        delta = (out_sel - v_mean).astype(bf16)
        return v_mean + _dot(p_sel, delta, (((0,), (0,))))

    for j in range(8):
        out_ref[0, :, j * D_K:(j + 1) * D_K] = _head(8 * h + j)


@jax.jit
def kernel(values, times, W_val, b_val, hour_emb, week_emb, month_emb,
           pos_emb, W_conv, b_conv, Wq, bq, Wk, bk, Wv, bv):
    tabs = jnp.zeros((N_TAB, D_MODEL), f32)
    tabs = tabs.at[0:24].set(hour_emb).at[24:31].set(week_emb).at[31:43].set(month_emb)
    args = (
        values.astype(bf16),
        times.astype(jnp.int32),
        pos_emb[0, :S, :],
        W_val.astype(bf16),
        tabs.astype(bf16),
        W_conv.astype(bf16),
        Wq.astype(bf16),
        Wk.astype(bf16),
        Wv.astype(bf16),
        b_val.reshape(1, D_MODEL),
        b_conv.reshape(1, D_MODEL),
        bq.reshape(1, D_MODEL),
        bk.reshape(1, D_MODEL),
        bv.reshape(1, D_MODEL),
    )
    full = lambda *dims: pl.BlockSpec(dims, lambda b, h: (0,) * len(dims))
    grid_spec = pltpu.PrefetchScalarGridSpec(
        num_scalar_prefetch=0,
        grid=(B, H // 8),
        in_specs=[
            pl.BlockSpec((1, S, D_IN), lambda b, h: (b, 0, 0)),
            pl.BlockSpec((1, S, 3), lambda b, h: (b, 0, 0)),
            full(S, D_MODEL),
            full(D_IN, D_MODEL),
            full(N_TAB, D_MODEL),
            full(3, D_MODEL, D_MODEL),
            full(D_MODEL, D_MODEL),
            full(D_MODEL, D_MODEL),
            full(D_MODEL, D_MODEL),
            full(1, D_MODEL),
            full(1, D_MODEL),
            full(1, D_MODEL),
            full(1, D_MODEL),
            full(1, D_MODEL),
        ],
        out_specs=pl.BlockSpec((1, S, 8 * D_K), lambda b, h: (b, 0, h)),
        scratch_shapes=[
            pltpu.VMEM((H, S, D_K), bf16),
            pltpu.VMEM((H, S, D_K), bf16),
            pltpu.VMEM((H, S, D_K), bf16),
            pltpu.VMEM((H, D_K), f32),
            pltpu.VMEM((H, S, D_K), bf16),
        ],
    )
    return pl.pallas_call(
        _body,
        grid_spec=grid_spec,
        out_shape=jax.ShapeDtypeStruct((B, S, D_MODEL), f32),
        compiler_params=pltpu.CompilerParams(
            dimension_semantics=("parallel", "arbitrary"),
        ),
    )(*args)


# speedup vs baseline: 1.0773x; 1.0188x over previous
import functools

import jax
import jax.numpy as jnp
from jax import lax
from jax.experimental import pallas as pl
from jax.experimental.pallas import tpu as pltpu

B, S, D_IN = 16, 512, 32
D_MODEL, H, D_K = 1024, 16, 64
U_TOP = 31
N_TAB = 48

f32 = jnp.float32
bf16 = jnp.bfloat16


def _dot(a, b, dims):
    return lax.dot_general(a, b, (dims, ((), ())), preferred_element_type=f32)


def _body(values_ref, times_ref, pos_ref, wval_ref, tabs_ref, wc_ref,
          wq_ref, wk_ref, wv_ref, bval_ref, bconv_ref, bq_ref, bk_ref, bv_ref,
          out_ref, q_scr, k_scr, v_scr, vmean_scr, kc_scr):
    h = pl.program_id(1)

    @pl.when(h == 0)
    def _qkv():
        col = lax.broadcasted_iota(jnp.int32, (S, N_TAB), 1)
        t0 = times_ref[0, :, 0:1]
        t1 = times_ref[0, :, 1:2]
        t2 = times_ref[0, :, 2:3]
        oh = ((col == t0) | (col == t1 + 24) | (col == t2 + 31)).astype(bf16)
        time_emb = _dot(oh, tabs_ref[...], (((1,), (0,))))
        value_emb = _dot(values_ref[0], wval_ref[...], (((1,), (0,))))
        combined = value_emb + bval_ref[...] + pos_ref[...] + time_emb
        cb = combined.astype(bf16)
        zrow = jnp.zeros((1, D_MODEL), bf16)
        c_prev = jnp.concatenate([zrow, cb[:-1]], axis=0)
        c_next = jnp.concatenate([cb[1:], zrow], axis=0)
        x = (_dot(c_prev, wc_ref[0], (((1,), (0,))))
             + _dot(cb, wc_ref[1], (((1,), (0,))))
             + _dot(c_next, wc_ref[2], (((1,), (0,))))
             + bconv_ref[...])
        x = jnp.where(x > 0, x, jnp.exp(x) - 1.0)
        xb = x.astype(bf16)
        q = _dot(xb, wq_ref[...], (((1,), (0,)))) + bq_ref[...]
        k = _dot(xb, wk_ref[...], (((1,), (0,)))) + bk_ref[...]
        v = _dot(xb, wv_ref[...], (((1,), (0,)))) + bv_ref[...]
        for hh in range(H):
            sl = slice(hh * D_K, (hh + 1) * D_K)
            q_scr[hh] = q[:, sl].astype(bf16)
            k_scr[hh] = k[:, sl].astype(bf16)
            v_scr[hh] = v[:, sl].astype(bf16)
            vmean_scr[hh:hh + 1] = jnp.mean(v[:, sl], axis=0, keepdims=True)
            kc_scr[hh] = (k[:, sl]
                          - jnp.mean(k[:, sl], axis=0, keepdims=True)).astype(bf16)

    def _head(hh):
        qb = q_scr[hh]
        kb = k_scr[hh]
        vb = v_scr[hh]

        s_cen = _dot(qb, kc_scr[hh], (((1,), (1,))))
        m_col = jnp.max(s_cen, axis=1, keepdims=True)
        m_row = jnp.transpose(m_col, (1, 0))
        c_gt = (m_col > m_row).astype(f32)
        rank_row = jnp.sum(c_gt, axis=0, keepdims=True)
        p_iota = lax.broadcasted_iota(jnp.int32, (U_TOP + 1, S), 0).astype(f32)
        p_sel = ((rank_row == p_iota)
                 & (p_iota < float(U_TOP))).astype(bf16)

        top_q = _dot(p_sel, qb, (((1,), (0,))))
        s_sel = _dot(top_q.astype(bf16), kb, (((1,), (1,)))) * (1.0 / 8.0)
        s_sel = s_sel - jnp.max(s_sel, axis=1, keepdims=True)
        e_sel = jnp.exp(s_sel)
        attn = e_sel * (1.0 / jnp.sum(e_sel, axis=1, keepdims=True))
        out_sel = _dot(attn.astype(bf16), vb, (((1,), (0,))))

        v_mean = vmean_scr[pl.ds(hh, 1)]
        delta = (out_sel - v_mean).astype(bf16)
        return v_mean + _dot(p_sel, delta, (((0,), (0,))))

    for j in range(16):
        out_ref[0, :, j * D_K:(j + 1) * D_K] = _head(16 * h + j)


@jax.jit
def kernel(values, times, W_val, b_val, hour_emb, week_emb, month_emb,
           pos_emb, W_conv, b_conv, Wq, bq, Wk, bk, Wv, bv):
    tabs = jnp.zeros((N_TAB, D_MODEL), f32)
    tabs = tabs.at[0:24].set(hour_emb).at[24:31].set(week_emb).at[31:43].set(month_emb)
    args = (
        values.astype(bf16),
        times.astype(jnp.int32),
        pos_emb[0, :S, :],
        W_val.astype(bf16),
        tabs.astype(bf16),
        W_conv.astype(bf16),
        Wq.astype(bf16),
        Wk.astype(bf16),
        Wv.astype(bf16),
        b_val.reshape(1, D_MODEL),
        b_conv.reshape(1, D_MODEL),
        bq.reshape(1, D_MODEL),
        bk.reshape(1, D_MODEL),
        bv.reshape(1, D_MODEL),
    )
    full = lambda *dims: pl.BlockSpec(dims, lambda b, h: (0,) * len(dims))
    grid_spec = pltpu.PrefetchScalarGridSpec(
        num_scalar_prefetch=0,
        grid=(B, H // 16),
        in_specs=[
            pl.BlockSpec((1, S, D_IN), lambda b, h: (b, 0, 0)),
            pl.BlockSpec((1, S, 3), lambda b, h: (b, 0, 0)),
            full(S, D_MODEL),
            full(D_IN, D_MODEL),
            full(N_TAB, D_MODEL),
            full(3, D_MODEL, D_MODEL),
            full(D_MODEL, D_MODEL),
            full(D_MODEL, D_MODEL),
            full(D_MODEL, D_MODEL),
            full(1, D_MODEL),
            full(1, D_MODEL),
            full(1, D_MODEL),
            full(1, D_MODEL),
            full(1, D_MODEL),
        ],
        out_specs=pl.BlockSpec((1, S, 16 * D_K), lambda b, h: (b, 0, h)),
        scratch_shapes=[
            pltpu.VMEM((H, S, D_K), bf16),
            pltpu.VMEM((H, S, D_K), bf16),
            pltpu.VMEM((H, S, D_K), bf16),
            pltpu.VMEM((H, D_K), f32),
            pltpu.VMEM((H, S, D_K), bf16),
        ],
    )
    return pl.pallas_call(
        _body,
        grid_spec=grid_spec,
        out_shape=jax.ShapeDtypeStruct((B, S, D_MODEL), f32),
        compiler_params=pltpu.CompilerParams(
            dimension_semantics=("parallel", "arbitrary"),
        ),
    )(*args)
